# baseline (speedup 1.0000x reference)
import functools

import jax
import jax.numpy as jnp
from jax import lax
from jax.experimental import pallas as pl
from jax.experimental.pallas import tpu as pltpu
from jax.experimental.pallas import tpu_sc as plsc

_NC = 2
_NS = 16
_L = 16
_NW = _NC * _NS


def _forward_process_sc(t, table_a, table_b, x_0, noise):
    B, C, L = x_0.shape
    T = table_a.shape[0]
    per_w = B // _NW
    lanes_per_row = C * L // _L

    mesh = plsc.VectorSubcoreMesh(core_axis_name="c", subcore_axis_name="s")

    @functools.partial(
        pl.kernel,
        out_type=jax.ShapeDtypeStruct((B, C, L), jnp.float32),
        mesh=mesh,
        compiler_params=pltpu.CompilerParams(needs_layout_passes=False),
        scratch_types=[
            pltpu.VMEM((per_w,), jnp.int32),
            pltpu.VMEM((T,), jnp.float32),
            pltpu.VMEM((T,), jnp.float32),
            pltpu.VMEM((per_w,), jnp.float32),
            pltpu.VMEM((per_w,), jnp.float32),
            pltpu.VMEM((1, C, L), jnp.float32),
            pltpu.VMEM((1, C, L), jnp.float32),
            pltpu.VMEM((1, C, L), jnp.float32),
            pltpu.VMEM((1, C, L), jnp.float32),
            pltpu.VMEM((1, C, L), jnp.float32),
            pltpu.VMEM((1, C, L), jnp.float32),
            pltpu.SemaphoreType.DMA,
            pltpu.SemaphoreType.DMA,
            pltpu.SemaphoreType.DMA,
            pltpu.SemaphoreType.DMA,
            pltpu.SemaphoreType.DMA,
            pltpu.SemaphoreType.DMA,
        ],
    )
    def sc_kernel(t_hbm, a_hbm, b_hbm, x_hbm, n_hbm, out_hbm,
                  idx_v, at_v, bt_v, ca_v, cb_v,
                  xv0, xv1, nv0, nv1, ov0, ov1,
                  sx0, sx1, sn0, sn1, so0, so1):
        wid = lax.axis_index("s") * _NC + lax.axis_index("c")
        base = wid * per_w

        pltpu.sync_copy(t_hbm.at[pl.ds(base, per_w)], idx_v)
        pltpu.sync_copy(a_hbm, at_v)
        pltpu.sync_copy(b_hbm, bt_v)
        for i in range(per_w // _L):
            sl = pl.ds(i * _L, _L)
            iv = idx_v[sl]
            ca_v[sl] = plsc.load_gather(at_v, [iv])
            cb_v[sl] = plsc.load_gather(bt_v, [iv])

        def in_start(r, xv, nv, sx, sn):
            src = x_hbm.at[pl.ds(base + r, 1)]
            pltpu.make_async_copy(src, xv, sx).start()
            src = n_hbm.at[pl.ds(base + r, 1)]
            pltpu.make_async_copy(src, nv, sn).start()

        def in_wait(r, xv, nv, sx, sn):
            src = x_hbm.at[pl.ds(base + r, 1)]
            pltpu.make_async_copy(src, xv, sx).wait()
            src = n_hbm.at[pl.ds(base + r, 1)]
            pltpu.make_async_copy(src, nv, sn).wait()

        def out_start(r, ov, so):
            dst = out_hbm.at[pl.ds(base + r, 1)]
            pltpu.make_async_copy(ov, dst, so).start()

        def out_wait(r, ov, so):
            dst = out_hbm.at[pl.ds(base + r, 1)]
            pltpu.make_async_copy(ov, dst, so).wait()

        def compute(r, xv, nv, ov):
            iv = jnp.full((_L,), r, jnp.int32)
            asp = plsc.load_gather(ca_v, [iv])
            bsp = plsc.load_gather(cb_v, [iv])

            @plsc.parallel_loop(0, C)
            def _srow(i):
                @plsc.parallel_loop(0, L // _L, unroll=8)
                def _schunk(j):
                    sl = pl.ds(j * _L, _L)
                    ov[0, i, sl] = asp * xv[0, i, sl] + bsp * nv[0, i, sl]

        n_iter = per_w // 2
        in_start(0, xv0, nv0, sx0, sn0)
        in_start(1, xv1, nv1, sx1, sn1)

        def ring_body(i, carry):
            r0 = 2 * i
            r1 = r0 + 1

            in_wait(r0, xv0, nv0, sx0, sn0)

            @pl.when(i > 0)
            def _():
                out_wait(r0 - 2, ov0, so0)

            compute(r0, xv0, nv0, ov0)
            out_start(r0, ov0, so0)

            @pl.when(i < n_iter - 1)
            def _():
                in_start(r0 + 2, xv0, nv0, sx0, sn0)

            in_wait(r1, xv1, nv1, sx1, sn1)

            @pl.when(i > 0)
            def _():
                out_wait(r1 - 2, ov1, so1)

            compute(r1, xv1, nv1, ov1)
            out_start(r1, ov1, so1)

            @pl.when(i < n_iter - 1)
            def _():
                in_start(r1 + 2, xv1, nv1, sx1, sn1)

            return carry

        lax.fori_loop(0, n_iter, ring_body, 0)
        out_wait(per_w - 2, ov0, so0)
        out_wait(per_w - 1, ov1, so1)

    return sc_kernel(t, table_a, table_b, x_0, noise)


def kernel(x_0, t, sqrt_alphas_cumprod, sqrt_one_minus_alphas_cumprod, noise):
    xt = _forward_process_sc(
        t, sqrt_alphas_cumprod, sqrt_one_minus_alphas_cumprod, x_0, noise)
    return (xt, noise)


# speedup vs baseline: 1.0041x; 1.0041x over previous
import functools

import jax
import jax.numpy as jnp
from jax import lax
from jax.experimental import pallas as pl
from jax.experimental.pallas import tpu as pltpu
from jax.experimental.pallas import tpu_sc as plsc

_NC = 2
_NS = 16
_L = 16
_NW = _NC * _NS


def _forward_process_sc(t, table_a, table_b, x_0, noise):
    B, C, L = x_0.shape
    T = table_a.shape[0]
    per_w = B // _NW
    lanes_per_row = C * L // _L

    mesh = plsc.VectorSubcoreMesh(core_axis_name="c", subcore_axis_name="s")

    @functools.partial(
        pl.kernel,
        out_type=jax.ShapeDtypeStruct((B, C, L), jnp.float32),
        mesh=mesh,
        compiler_params=pltpu.CompilerParams(
            needs_layout_passes=False, use_tc_tiling_on_sc=True),
        scratch_types=[
            pltpu.VMEM((per_w,), jnp.int32),
            pltpu.VMEM((T,), jnp.float32),
            pltpu.VMEM((T,), jnp.float32),
            pltpu.VMEM((per_w,), jnp.float32),
            pltpu.VMEM((per_w,), jnp.float32),
            pltpu.VMEM((1, C, L), jnp.float32),
            pltpu.VMEM((1, C, L), jnp.float32),
            pltpu.VMEM((1, C, L), jnp.float32),
            pltpu.VMEM((1, C, L), jnp.float32),
            pltpu.VMEM((1, C, L), jnp.float32),
            pltpu.VMEM((1, C, L), jnp.float32),
            pltpu.SemaphoreType.DMA,
            pltpu.SemaphoreType.DMA,
            pltpu.SemaphoreType.DMA,
            pltpu.SemaphoreType.DMA,
            pltpu.SemaphoreType.DMA,
            pltpu.SemaphoreType.DMA,
        ],
    )
    def sc_kernel(t_hbm, a_hbm, b_hbm, x_hbm, n_hbm, out_hbm,
                  idx_v, at_v, bt_v, ca_v, cb_v,
                  xv0, xv1, nv0, nv1, ov0, ov1,
                  sx0, sx1, sn0, sn1, so0, so1):
        wid = lax.axis_index("s") * _NC + lax.axis_index("c")
        base = wid * per_w

        pltpu.sync_copy(t_hbm.at[pl.ds(base, per_w)], idx_v)
        pltpu.sync_copy(a_hbm, at_v)
        pltpu.sync_copy(b_hbm, bt_v)
        for i in range(per_w // _L):
            sl = pl.ds(i * _L, _L)
            iv = idx_v[sl]
            ca_v[sl] = plsc.load_gather(at_v, [iv])
            cb_v[sl] = plsc.load_gather(bt_v, [iv])

        def in_start(r, xv, nv, sx, sn):
            src = x_hbm.at[pl.ds(base + r, 1)]
            pltpu.make_async_copy(src, xv, sx).start()
            src = n_hbm.at[pl.ds(base + r, 1)]
            pltpu.make_async_copy(src, nv, sn).start()

        def in_wait(r, xv, nv, sx, sn):
            src = x_hbm.at[pl.ds(base + r, 1)]
            pltpu.make_async_copy(src, xv, sx).wait()
            src = n_hbm.at[pl.ds(base + r, 1)]
            pltpu.make_async_copy(src, nv, sn).wait()

        def out_start(r, ov, so):
            dst = out_hbm.at[pl.ds(base + r, 1)]
            pltpu.make_async_copy(ov, dst, so).start()

        def out_wait(r, ov, so):
            dst = out_hbm.at[pl.ds(base + r, 1)]
            pltpu.make_async_copy(ov, dst, so).wait()

        def compute(r, xv, nv, ov):
            iv = jnp.full((_L,), r, jnp.int32)
            asp = plsc.load_gather(ca_v, [iv])
            bsp = plsc.load_gather(cb_v, [iv])

            @plsc.parallel_loop(0, C)
            def _srow(i):
                @plsc.parallel_loop(0, L // _L, unroll=8)
                def _schunk(j):
                    sl = pl.ds(j * _L, _L)
                    ov[0, i, sl] = asp * xv[0, i, sl] + bsp * nv[0, i, sl]

        n_iter = per_w // 2
        in_start(0, xv0, nv0, sx0, sn0)
        in_start(1, xv1, nv1, sx1, sn1)

        def ring_body(i, carry):
            r0 = 2 * i
            r1 = r0 + 1

            in_wait(r0, xv0, nv0, sx0, sn0)

            @pl.when(i > 0)
            def _():
                out_wait(r0 - 2, ov0, so0)

            compute(r0, xv0, nv0, ov0)
            out_start(r0, ov0, so0)

            @pl.when(i < n_iter - 1)
            def _():
                in_start(r0 + 2, xv0, nv0, sx0, sn0)

            in_wait(r1, xv1, nv1, sx1, sn1)

            @pl.when(i > 0)
            def _():
                out_wait(r1 - 2, ov1, so1)

            compute(r1, xv1, nv1, ov1)
            out_start(r1, ov1, so1)

            @pl.when(i < n_iter - 1)
            def _():
                in_start(r1 + 2, xv1, nv1, sx1, sn1)

            return carry

        lax.fori_loop(0, n_iter, ring_body, 0)
        out_wait(per_w - 2, ov0, so0)
        out_wait(per_w - 1, ov1, so1)

    return sc_kernel(t, table_a, table_b, x_0, noise)


def kernel(x_0, t, sqrt_alphas_cumprod, sqrt_one_minus_alphas_cumprod, noise):
    xt = _forward_process_sc(
        t, sqrt_alphas_cumprod, sqrt_one_minus_alphas_cumprod, x_0, noise)
    return (xt, noise)


# speedup vs baseline: 1.1574x; 1.1527x over previous
import functools

import jax
import jax.numpy as jnp
from jax import lax
from jax.experimental import pallas as pl
from jax.experimental.pallas import tpu as pltpu
from jax.experimental.pallas import tpu_sc as plsc

_NC = 2
_NS = 16
_L = 16
_NW = _NC * _NS


def _forward_process_sc(t, table_a, table_b, x_0, noise):
    B, C, L = x_0.shape
    T = table_a.shape[0]
    per_w = B // _NW
    lanes_per_row = C * L // _L

    mesh = plsc.VectorSubcoreMesh(core_axis_name="c", subcore_axis_name="s")

    @functools.partial(
        pl.kernel,
        out_type=[
            jax.ShapeDtypeStruct((B, C, L), jnp.float32),
            jax.ShapeDtypeStruct((B, C, L), jnp.float32),
        ],
        mesh=mesh,
        compiler_params=pltpu.CompilerParams(
            needs_layout_passes=False, use_tc_tiling_on_sc=True),
        scratch_types=[
            pltpu.VMEM((per_w,), jnp.int32),
            pltpu.VMEM((T,), jnp.float32),
            pltpu.VMEM((T,), jnp.float32),
            pltpu.VMEM((per_w,), jnp.float32),
            pltpu.VMEM((per_w,), jnp.float32),
            pltpu.VMEM((1, C, L), jnp.float32),
            pltpu.VMEM((1, C, L), jnp.float32),
            pltpu.VMEM((1, C, L), jnp.float32),
            pltpu.VMEM((1, C, L), jnp.float32),
            pltpu.VMEM((1, C, L), jnp.float32),
            pltpu.VMEM((1, C, L), jnp.float32),
            pltpu.SemaphoreType.DMA,
            pltpu.SemaphoreType.DMA,
            pltpu.SemaphoreType.DMA,
            pltpu.SemaphoreType.DMA,
            pltpu.SemaphoreType.DMA,
            pltpu.SemaphoreType.DMA,
            pltpu.SemaphoreType.DMA,
            pltpu.SemaphoreType.DMA,
        ],
    )
    def sc_kernel(t_hbm, a_hbm, b_hbm, x_hbm, n_hbm, out_hbm, nout_hbm,
                  idx_v, at_v, bt_v, ca_v, cb_v,
                  xv0, xv1, nv0, nv1, ov0, ov1,
                  sx0, sx1, sn0, sn1, so0, so1, sno0, sno1):
        wid = lax.axis_index("s") * _NC + lax.axis_index("c")
        base = wid * per_w

        pltpu.sync_copy(t_hbm.at[pl.ds(base, per_w)], idx_v)
        pltpu.sync_copy(a_hbm, at_v)
        pltpu.sync_copy(b_hbm, bt_v)
        for i in range(per_w // _L):
            sl = pl.ds(i * _L, _L)
            iv = idx_v[sl]
            ca_v[sl] = plsc.load_gather(at_v, [iv])
            cb_v[sl] = plsc.load_gather(bt_v, [iv])

        def in_start(r, xv, nv, sx, sn):
            src = x_hbm.at[pl.ds(base + r, 1)]
            pltpu.make_async_copy(src, xv, sx).start()
            src = n_hbm.at[pl.ds(base + r, 1)]
            pltpu.make_async_copy(src, nv, sn).start()

        def in_wait(r, xv, nv, sx, sn):
            src = x_hbm.at[pl.ds(base + r, 1)]
            pltpu.make_async_copy(src, xv, sx).wait()
            src = n_hbm.at[pl.ds(base + r, 1)]
            pltpu.make_async_copy(src, nv, sn).wait()

        def out_start(r, ov, so):
            dst = out_hbm.at[pl.ds(base + r, 1)]
            pltpu.make_async_copy(ov, dst, so).start()

        def out_wait(r, ov, so):
            dst = out_hbm.at[pl.ds(base + r, 1)]
            pltpu.make_async_copy(ov, dst, so).wait()

        def nout_start(r, nv, sno):
            dst = nout_hbm.at[pl.ds(base + r, 1)]
            pltpu.make_async_copy(nv, dst, sno).start()

        def nout_wait(r, nv, sno):
            dst = nout_hbm.at[pl.ds(base + r, 1)]
            pltpu.make_async_copy(nv, dst, sno).wait()

        def compute(r, xv, nv, ov):
            iv = jnp.full((_L,), r, jnp.int32)
            asp = plsc.load_gather(ca_v, [iv])
            bsp = plsc.load_gather(cb_v, [iv])

            @plsc.parallel_loop(0, C)
            def _srow(i):
                @plsc.parallel_loop(0, L // _L, unroll=8)
                def _schunk(j):
                    sl = pl.ds(j * _L, _L)
                    ov[0, i, sl] = asp * xv[0, i, sl] + bsp * nv[0, i, sl]

        n_iter = per_w // 2
        in_start(0, xv0, nv0, sx0, sn0)
        in_start(1, xv1, nv1, sx1, sn1)

        def ring_body(i, carry):
            r0 = 2 * i
            r1 = r0 + 1

            in_wait(r0, xv0, nv0, sx0, sn0)
            nout_start(r0, nv0, sno0)

            @pl.when(i > 0)
            def _():
                out_wait(r0 - 2, ov0, so0)

            compute(r0, xv0, nv0, ov0)
            out_start(r0, ov0, so0)

            @pl.when(i < n_iter - 1)
            def _():
                nout_wait(r0, nv0, sno0)
                in_start(r0 + 2, xv0, nv0, sx0, sn0)

            in_wait(r1, xv1, nv1, sx1, sn1)
            nout_start(r1, nv1, sno1)

            @pl.when(i > 0)
            def _():
                out_wait(r1 - 2, ov1, so1)

            compute(r1, xv1, nv1, ov1)
            out_start(r1, ov1, so1)

            @pl.when(i < n_iter - 1)
            def _():
                nout_wait(r1, nv1, sno1)
                in_start(r1 + 2, xv1, nv1, sx1, sn1)

            return carry

        lax.fori_loop(0, n_iter, ring_body, 0)
        out_wait(per_w - 2, ov0, so0)
        out_wait(per_w - 1, ov1, so1)
        nout_wait(per_w - 2, nv0, sno0)
        nout_wait(per_w - 1, nv1, sno1)

    return sc_kernel(t, table_a, table_b, x_0, noise)


def kernel(x_0, t, sqrt_alphas_cumprod, sqrt_one_minus_alphas_cumprod, noise):
    xt, nout = _forward_process_sc(
        t, sqrt_alphas_cumprod, sqrt_one_minus_alphas_cumprod, x_0, noise)
    return (xt, nout)
